# baseline (speedup 1.0000x reference)
import functools

import jax
import jax.numpy as jnp
from jax.experimental import pallas as pl


def _select_kernel(x_ref, p_ref, b_ref, g_ref, v_ref):
    x = x_ref[0]
    p_full = p_ref[...]
    bias = b_ref[0, 0]

    y = jax.lax.dot_general(
        x.astype(jnp.bfloat16), p_full.astype(jnp.bfloat16),
        (((1,), (0,)), ((), ())),
        preferred_element_type=jnp.float32)
    wcol = y[:, 0:1]
    s_col = jax.nn.sigmoid(wcol + bias)
    s_row = s_col.T

    jlt = (jax.lax.broadcasted_iota(jnp.int32, (512, 512), 0)
           < jax.lax.broadcasted_iota(jnp.int32, (512, 512), 1))
    cmp = (s_col > s_row) | ((s_col == s_row) & jlt)
    rank_row = jnp.sum(cmp.astype(jnp.int32), axis=0,
                       keepdims=True)

    r_iota = jax.lax.broadcasted_iota(jnp.int32, (128, 512), 0)
    m = (rank_row == r_iota).astype(jnp.float32)

    g_ref[0] = jax.lax.dot_general(
        m, x, (((1,), (0,)), ((), ())),
        preferred_element_type=jnp.float32,
        precision=jax.lax.Precision.HIGHEST)
    li = jax.lax.broadcasted_iota(jnp.int32, (512, 128), 1)
    s_pad = jnp.where(li == 0, s_col, jnp.float32(0.0))
    v_ref[0] = jax.lax.dot_general(
        m, s_pad, (((1,), (0,)), ((), ())),
        preferred_element_type=jnp.float32,
        precision=jax.lax.Precision.HIGHEST)[:, 0:1]


def _scale_kernel(g_ref, v_ref, o_ref):
    o_ref[0] = g_ref[0] * v_ref[...]


@jax.jit
def kernel(node_fts, rel_edges, W, b):
    del rel_edges
    B, N, D = node_fts.shape
    K = 128
    b2 = b.reshape(1, 1).astype(jnp.float32)
    p = jnp.pad(W.reshape(D, 1), ((0, 0), (0, D - 1)))

    gathered, vals = pl.pallas_call(
        _select_kernel,
        grid=(B,),
        in_specs=[
            pl.BlockSpec((1, N, D), lambda i: (i, 0, 0)),
            pl.BlockSpec((D, D), lambda i: (0, 0)),
            pl.BlockSpec((1, 1), lambda i: (0, 0)),
        ],
        out_specs=[
            pl.BlockSpec((1, K, D), lambda i: (i, 0, 0)),
            pl.BlockSpec((1, K, 1), lambda i: (i, 0, 0)),
        ],
        out_shape=[
            jax.ShapeDtypeStruct((B, K, D), jnp.float32),
            jax.ShapeDtypeStruct((B, K, 1), jnp.float32),
        ],
    )(node_fts, p, b2)

    value = vals.reshape(B, K)

    out = pl.pallas_call(
        _scale_kernel,
        grid=(B,),
        in_specs=[
            pl.BlockSpec((1, K, D), lambda i: (i, 0, 0)),
            pl.BlockSpec((K, D), lambda i: (0, 0)),
        ],
        out_specs=pl.BlockSpec((1, K, D), lambda i: (i, 0, 0)),
        out_shape=jax.ShapeDtypeStruct((B, K, D), jnp.float32),
    )(gathered, value)
    return out


# speedup vs baseline: 1.2825x; 1.2825x over previous
import functools

import jax
import jax.numpy as jnp
from jax.experimental import pallas as pl


def _select_kernel(x_ref, p_ref, b_ref, g_ref, v_ref):
    x = x_ref[0]
    p_full = p_ref[...]
    bias = b_ref[0, 0]

    y = jax.lax.dot_general(
        x.astype(jnp.bfloat16), p_full.astype(jnp.bfloat16),
        (((1,), (0,)), ((), ())),
        preferred_element_type=jnp.float32)
    wcol = y[:, 0:1]
    s_col = jax.nn.sigmoid(wcol + bias)
    s_row = s_col.T

    rank_row = jnp.zeros((1, 512), jnp.int32)
    i_iota = jax.lax.broadcasted_iota(jnp.int32, (64, 512), 1)
    for c in range(8):
        s_c = jax.lax.slice(s_col, (64 * c, 0), (64 * c + 64, 1))
        j_iota = jax.lax.broadcasted_iota(jnp.int32, (64, 512), 0) + (64 * c)
        cmp = (s_c > s_row) | ((s_c == s_row) & (j_iota < i_iota))
        rank_row = rank_row + jnp.sum(cmp.astype(jnp.int32), axis=0,
                                      keepdims=True)

    r_iota = jax.lax.broadcasted_iota(jnp.int32, (128, 512), 0)
    m = (rank_row == r_iota).astype(jnp.float32)

    g_ref[0] = jax.lax.dot_general(
        m, x, (((1,), (0,)), ((), ())),
        preferred_element_type=jnp.float32,
        precision=jax.lax.Precision.HIGHEST)
    li = jax.lax.broadcasted_iota(jnp.int32, (512, 128), 1)
    s_pad = jnp.where(li == 0, s_col, jnp.float32(0.0))
    v_ref[0] = jax.lax.dot_general(
        m, s_pad, (((1,), (0,)), ((), ())),
        preferred_element_type=jnp.float32,
        precision=jax.lax.Precision.HIGHEST)[:, 0:1]


def _scale_kernel(g_ref, v_ref, o_ref):
    o_ref[...] = g_ref[...] * v_ref[...][None, :, :]


@jax.jit
def kernel(node_fts, rel_edges, W, b):
    del rel_edges
    B, N, D = node_fts.shape
    K = 128
    b2 = b.reshape(1, 1).astype(jnp.float32)
    p = jnp.pad(W.reshape(D, 1), ((0, 0), (0, D - 1)))

    gathered, vals = pl.pallas_call(
        _select_kernel,
        grid=(B,),
        in_specs=[
            pl.BlockSpec((1, N, D), lambda i: (i, 0, 0)),
            pl.BlockSpec((D, D), lambda i: (0, 0)),
            pl.BlockSpec((1, 1), lambda i: (0, 0)),
        ],
        out_specs=[
            pl.BlockSpec((1, K, D), lambda i: (i, 0, 0)),
            pl.BlockSpec((1, K, 1), lambda i: (i, 0, 0)),
        ],
        out_shape=[
            jax.ShapeDtypeStruct((B, K, D), jnp.float32),
            jax.ShapeDtypeStruct((B, K, 1), jnp.float32),
        ],
    )(node_fts, p, b2)

    value = vals.reshape(B, K)

    SB = 16
    out = pl.pallas_call(
        _scale_kernel,
        grid=(B // SB,),
        in_specs=[
            pl.BlockSpec((SB, K, D), lambda i: (i, 0, 0)),
            pl.BlockSpec((K, D), lambda i: (0, 0)),
        ],
        out_specs=pl.BlockSpec((SB, K, D), lambda i: (i, 0, 0)),
        out_shape=jax.ShapeDtypeStruct((B, K, D), jnp.float32),
    )(gathered, value)
    return out


# speedup vs baseline: 1.2859x; 1.0027x over previous
import functools

import jax
import jax.numpy as jnp
from jax.experimental import pallas as pl
from jax.experimental.pallas import tpu as pltpu


def _select_kernel(x_ref, p_ref, b_ref, g_ref, v_ref):
    x = x_ref[0]
    p_full = p_ref[...]
    bias = b_ref[0, 0]

    y = jax.lax.dot_general(
        x.astype(jnp.bfloat16), p_full.astype(jnp.bfloat16),
        (((1,), (0,)), ((), ())),
        preferred_element_type=jnp.float32)
    wcol = y[:, 0:1]
    s_col = jax.nn.sigmoid(wcol + bias)
    s_row = s_col.T

    rank_row = jnp.zeros((1, 512), jnp.int32)
    i_iota = jax.lax.broadcasted_iota(jnp.int32, (64, 512), 1)
    for c in range(8):
        s_c = jax.lax.slice(s_col, (64 * c, 0), (64 * c + 64, 1))
        j_iota = jax.lax.broadcasted_iota(jnp.int32, (64, 512), 0) + (64 * c)
        cmp = (s_c > s_row) | ((s_c == s_row) & (j_iota < i_iota))
        rank_row = rank_row + jnp.sum(cmp.astype(jnp.int32), axis=0,
                                      keepdims=True)

    r_iota = jax.lax.broadcasted_iota(jnp.int32, (128, 512), 0)
    m = (rank_row == r_iota).astype(jnp.float32)

    g_ref[0] = jax.lax.dot_general(
        m, x, (((1,), (0,)), ((), ())),
        preferred_element_type=jnp.float32,
        precision=jax.lax.Precision.HIGHEST)
    li = jax.lax.broadcasted_iota(jnp.int32, (512, 128), 1)
    s_pad = jnp.where(li == 0, s_col, jnp.float32(0.0))
    v_ref[0] = jax.lax.dot_general(
        m, s_pad, (((1,), (0,)), ((), ())),
        preferred_element_type=jnp.float32,
        precision=jax.lax.Precision.HIGHEST)[:, 0:1]


def _scale_kernel(g_ref, v_ref, o_ref):
    o_ref[...] = g_ref[...] * v_ref[...][None, :, :]


@jax.jit
def kernel(node_fts, rel_edges, W, b):
    del rel_edges
    B, N, D = node_fts.shape
    K = 128
    b2 = b.reshape(1, 1).astype(jnp.float32)
    p = jnp.pad(W.reshape(D, 1), ((0, 0), (0, D - 1)))

    gathered, vals = pl.pallas_call(
        _select_kernel,
        grid=(B,),
        in_specs=[
            pl.BlockSpec((1, N, D), lambda i: (i, 0, 0)),
            pl.BlockSpec((D, D), lambda i: (0, 0)),
            pl.BlockSpec((1, 1), lambda i: (0, 0)),
        ],
        out_specs=[
            pl.BlockSpec((1, K, D), lambda i: (i, 0, 0)),
            pl.BlockSpec((1, K, 1), lambda i: (i, 0, 0)),
        ],
        out_shape=[
            jax.ShapeDtypeStruct((B, K, D), jnp.float32),
            jax.ShapeDtypeStruct((B, K, 1), jnp.float32),
        ],
        compiler_params=pltpu.CompilerParams(
            dimension_semantics=("parallel",)),
    )(node_fts, p, b2)

    value = vals.reshape(B, K)

    SB = 16
    out = pl.pallas_call(
        _scale_kernel,
        grid=(B // SB,),
        in_specs=[
            pl.BlockSpec((SB, K, D), lambda i: (i, 0, 0)),
            pl.BlockSpec((K, D), lambda i: (0, 0)),
        ],
        out_specs=pl.BlockSpec((SB, K, D), lambda i: (i, 0, 0)),
        out_shape=jax.ShapeDtypeStruct((B, K, D), jnp.float32),
        compiler_params=pltpu.CompilerParams(
            dimension_semantics=("parallel",)),
    )(gathered, value)
    return out


# speedup vs baseline: 1.4935x; 1.1614x over previous
import functools

import jax
import jax.numpy as jnp
from jax import lax
from jax.experimental import pallas as pl
from jax.experimental.pallas import tpu as pltpu
from jax.experimental.pallas import tpu_sc as plsc


def _score_rank_kernel(x_ref, p_ref, b_ref, r_ref, s_ref):
    x = x_ref[0]
    p_full = p_ref[...]
    bias = b_ref[0, 0]

    y = jax.lax.dot_general(
        x.astype(jnp.bfloat16), p_full.astype(jnp.bfloat16),
        (((1,), (0,)), ((), ())),
        preferred_element_type=jnp.float32)
    wcol = y[:, 0:1]
    s_col = jax.nn.sigmoid(wcol + bias)
    s_row = s_col.T

    rank_row = jnp.zeros((1, 512), jnp.int32)
    i_iota = jax.lax.broadcasted_iota(jnp.int32, (64, 512), 1)
    for c in range(8):
        s_c = jax.lax.slice(s_col, (64 * c, 0), (64 * c + 64, 1))
        j_iota = jax.lax.broadcasted_iota(jnp.int32, (64, 512), 0) + (64 * c)
        cmp = (s_c > s_row) | ((s_c == s_row) & (j_iota < i_iota))
        rank_row = rank_row + jnp.sum(cmp.astype(jnp.int32), axis=0,
                                      keepdims=True)

    r_ref[0] = rank_row
    s_ref[0] = s_row


def _sc_select_gather(rank_hbm, score_hbm, node_hbm,
                      gath_hbm, val_hbm,
                      rank_v, score_v, idx_v, vals_v, rows_v, sem):
    nc = 2
    wid = lax.axis_index("s") * nc + lax.axis_index("c")
    for t in range(4):
        b = wid * 4 + t
        pltpu.sync_copy(rank_hbm.at[b], rank_v)
        pltpu.sync_copy(score_hbm.at[b], score_v)
        for c in range(32):
            r16 = rank_v[pl.ds(c * 16, 16)]
            s16 = score_v[pl.ds(c * 16, 16)]
            i16 = lax.iota(jnp.int32, 16) + (c * 16)
            msk = r16 < 128
            plsc.store_scatter(idx_v, [r16], i16, mask=msk)
            plsc.store_scatter(vals_v, [r16], s16, mask=msk)
        pltpu.async_copy(node_hbm.at[b].at[idx_v], rows_v, sem).wait()
        pltpu.sync_copy(rows_v, gath_hbm.at[b])
        pltpu.sync_copy(vals_v, val_hbm.at[b])


def _scale_kernel(g_ref, v_ref, o_ref):
    o_ref[...] = g_ref[...] * v_ref[...][None, :, :]


@jax.jit
def kernel(node_fts, rel_edges, W, b):
    del rel_edges
    B, N, D = node_fts.shape
    K = 128
    b2 = b.reshape(1, 1).astype(jnp.float32)
    p = jnp.pad(W.reshape(D, 1), ((0, 0), (0, D - 1)))

    ranks, scores = pl.pallas_call(
        _score_rank_kernel,
        grid=(B,),
        in_specs=[
            pl.BlockSpec((1, N, D), lambda i: (i, 0, 0)),
            pl.BlockSpec((D, D), lambda i: (0, 0)),
            pl.BlockSpec((1, 1), lambda i: (0, 0)),
        ],
        out_specs=[
            pl.BlockSpec((1, 1, N), lambda i: (i, 0, 0)),
            pl.BlockSpec((1, 1, N), lambda i: (i, 0, 0)),
        ],
        out_shape=[
            jax.ShapeDtypeStruct((B, 1, N), jnp.int32),
            jax.ShapeDtypeStruct((B, 1, N), jnp.float32),
        ],
    )(node_fts, p, b2)
    ranks = ranks.reshape(B, N)
    scores = scores.reshape(B, N)

    sc_fn = functools.partial(
        pl.kernel,
        mesh=plsc.VectorSubcoreMesh(core_axis_name="c", subcore_axis_name="s"),
        out_type=[
            jax.ShapeDtypeStruct((B, K, D), jnp.float32),
            jax.ShapeDtypeStruct((B, K), jnp.float32),
        ],
        scratch_types=[
            pltpu.VMEM((N,), jnp.int32),
            pltpu.VMEM((N,), jnp.float32),
            pltpu.VMEM((K,), jnp.int32),
            pltpu.VMEM((K,), jnp.float32),
            pltpu.VMEM((K, D), jnp.float32),
            pltpu.SemaphoreType.DMA,
        ],
        compiler_params=pltpu.CompilerParams(needs_layout_passes=False),
    )(_sc_select_gather)
    gathered, value = sc_fn(ranks, scores, node_fts)

    SB = 16
    out = pl.pallas_call(
        _scale_kernel,
        grid=(B // SB,),
        in_specs=[
            pl.BlockSpec((SB, K, D), lambda i: (i, 0, 0)),
            pl.BlockSpec((K, D), lambda i: (0, 0)),
        ],
        out_specs=pl.BlockSpec((SB, K, D), lambda i: (i, 0, 0)),
        out_shape=jax.ShapeDtypeStruct((B, K, D), jnp.float32),
        compiler_params=pltpu.CompilerParams(
            dimension_semantics=("parallel",)),
    )(gathered, value)
    return out


# speedup vs baseline: 1.9505x; 1.3060x over previous
import functools

import jax
import jax.numpy as jnp
from jax import lax
from jax.experimental import pallas as pl
from jax.experimental.pallas import tpu as pltpu
from jax.experimental.pallas import tpu_sc as plsc


def _score_rank_kernel(x_ref, p_ref, b_ref, r_ref, s_ref):
    p_full = p_ref[...]
    bias = b_ref[0, 0]
    i_iota = jax.lax.broadcasted_iota(jnp.int32, (64, 512), 1)

    for t in range(x_ref.shape[0]):
        x = x_ref[t]
        y = jax.lax.dot_general(
            x.astype(jnp.bfloat16), p_full.astype(jnp.bfloat16),
            (((1,), (0,)), ((), ())),
            preferred_element_type=jnp.float32)
        wcol = y[:, 0:1]
        s_row = jax.nn.sigmoid(wcol.T + bias)
        s_col = s_row.T

        rank_row = jnp.zeros((1, 512), jnp.int32)
        for c in range(8):
            s_c = jax.lax.slice(s_col, (64 * c, 0), (64 * c + 64, 1))
            j_iota = (jax.lax.broadcasted_iota(jnp.int32, (64, 512), 0)
                      + (64 * c))
            cmp = (s_c > s_row) | ((s_c == s_row) & (j_iota < i_iota))
            rank_row = rank_row + jnp.sum(cmp.astype(jnp.int32), axis=0,
                                          keepdims=True)

        r_ref[t] = rank_row
        s_ref[t] = s_row


def _sc_select_gather(rank_hbm, score_hbm, node_hbm,
                      gath_hbm, val_hbm,
                      rank_v, score_v, idx_v, vals_v, rows_v, sem):
    nc = 2
    wid = lax.axis_index("s") * nc + lax.axis_index("c")
    for t in range(4):
        b = wid * 4 + t
        pltpu.sync_copy(rank_hbm.at[b], rank_v)
        pltpu.sync_copy(score_hbm.at[b], score_v)
        for c in range(32):
            r16 = rank_v[pl.ds(c * 16, 16)]
            s16 = score_v[pl.ds(c * 16, 16)]
            i16 = lax.iota(jnp.int32, 16) + (c * 16)
            msk = r16 < 128
            plsc.store_scatter(idx_v, [r16], i16, mask=msk)
            plsc.store_scatter(vals_v, [r16], s16, mask=msk)
        pltpu.async_copy(node_hbm.at[b].at[idx_v], rows_v, sem).wait()
        pltpu.sync_copy(rows_v, gath_hbm.at[b])
        pltpu.sync_copy(vals_v, val_hbm.at[b])


def _scale_kernel(g_ref, v_ref, o_ref):
    o_ref[...] = g_ref[...] * v_ref[...][None, :, :]


@jax.jit
def kernel(node_fts, rel_edges, W, b):
    del rel_edges
    B, N, D = node_fts.shape
    K = 128
    b2 = b.reshape(1, 1).astype(jnp.float32)
    p = jnp.pad(W.reshape(D, 1), ((0, 0), (0, D - 1)))

    TB = 2
    ranks, scores = pl.pallas_call(
        _score_rank_kernel,
        grid=(B // TB,),
        in_specs=[
            pl.BlockSpec((TB, N, D), lambda i: (i, 0, 0)),
            pl.BlockSpec((D, D), lambda i: (0, 0)),
            pl.BlockSpec((1, 1), lambda i: (0, 0)),
        ],
        out_specs=[
            pl.BlockSpec((TB, 1, N), lambda i: (i, 0, 0)),
            pl.BlockSpec((TB, 1, N), lambda i: (i, 0, 0)),
        ],
        out_shape=[
            jax.ShapeDtypeStruct((B, 1, N), jnp.int32),
            jax.ShapeDtypeStruct((B, 1, N), jnp.float32),
        ],
    )(node_fts, p, b2)
    ranks = ranks.reshape(B, N)
    scores = scores.reshape(B, N)

    sc_fn = functools.partial(
        pl.kernel,
        mesh=plsc.VectorSubcoreMesh(core_axis_name="c", subcore_axis_name="s"),
        out_type=[
            jax.ShapeDtypeStruct((B, K, D), jnp.float32),
            jax.ShapeDtypeStruct((B, K), jnp.float32),
        ],
        scratch_types=[
            pltpu.VMEM((N,), jnp.int32),
            pltpu.VMEM((N,), jnp.float32),
            pltpu.VMEM((K,), jnp.int32),
            pltpu.VMEM((K,), jnp.float32),
            pltpu.VMEM((K, D), jnp.float32),
            pltpu.SemaphoreType.DMA,
        ],
        compiler_params=pltpu.CompilerParams(needs_layout_passes=False),
    )(_sc_select_gather)
    gathered, value = sc_fn(ranks, scores, node_fts)

    SB = 16
    out = pl.pallas_call(
        _scale_kernel,
        grid=(B // SB,),
        in_specs=[
            pl.BlockSpec((SB, K, D), lambda i: (i, 0, 0)),
            pl.BlockSpec((K, D), lambda i: (0, 0)),
        ],
        out_specs=pl.BlockSpec((SB, K, D), lambda i: (i, 0, 0)),
        out_shape=jax.ShapeDtypeStruct((B, K, D), jnp.float32),
        compiler_params=pltpu.CompilerParams(
            dimension_semantics=("parallel",)),
    )(gathered, value)
    return out


# speedup vs baseline: 2.1544x; 1.1045x over previous
import functools

import jax
import jax.numpy as jnp
from jax import lax
from jax.experimental import pallas as pl
from jax.experimental.pallas import tpu as pltpu
from jax.experimental.pallas import tpu_sc as plsc


def _score_rank_kernel(x_ref, p_ref, b_ref, r_ref, s_ref):
    p_full = p_ref[...]
    bias = b_ref[0, 0]
    i_iota = jax.lax.broadcasted_iota(jnp.int32, (64, 512), 1)

    for t in range(x_ref.shape[0]):
        x = x_ref[t]
        y = jax.lax.dot_general(
            x.astype(jnp.bfloat16), p_full.astype(jnp.bfloat16),
            (((1,), (0,)), ((), ())),
            preferred_element_type=jnp.float32)
        wcol = y[:, 0:1]
        s_row = jax.nn.sigmoid(wcol.T + bias)
        s_col = s_row.T

        rank_row = jnp.zeros((1, 512), jnp.int32)
        for c in range(8):
            s_c = jax.lax.slice(s_col, (64 * c, 0), (64 * c + 64, 1))
            j_iota = (jax.lax.broadcasted_iota(jnp.int32, (64, 512), 0)
                      + (64 * c))
            cmp = (s_c > s_row) | ((s_c == s_row) & (j_iota < i_iota))
            rank_row = rank_row + jnp.sum(cmp.astype(jnp.int32), axis=0,
                                          keepdims=True)

        r_ref[t] = rank_row
        s_ref[t] = s_row


def _sc_select_gather(rank_hbm, score_hbm, node_hbm,
                      gath_hbm, val_hbm,
                      rank_v, score_v, idx_v, vals_v, rows_v, sem):
    nc = 2
    wid = lax.axis_index("s") * nc + lax.axis_index("c")
    for t in range(4):
        b = wid * 4 + t
        pltpu.sync_copy(rank_hbm.at[b], rank_v)
        pltpu.sync_copy(score_hbm.at[b], score_v)
        for c in range(32):
            r16 = rank_v[pl.ds(c * 16, 16)]
            s16 = score_v[pl.ds(c * 16, 16)]
            i16 = lax.iota(jnp.int32, 16) + (c * 16)
            msk = r16 < 128
            plsc.store_scatter(idx_v, [r16], i16, mask=msk)
            plsc.store_scatter(vals_v, [r16], s16, mask=msk)
        pltpu.async_copy(node_hbm.at[b].at[idx_v], rows_v, sem).wait()
        pltpu.sync_copy(rows_v, gath_hbm.at[b])
        pltpu.sync_copy(vals_v, val_hbm.at[b])


def _scale_kernel(g_ref, v_ref, o_ref):
    o_ref[...] = g_ref[...] * v_ref[...][None, :, :]


@jax.jit
def kernel(node_fts, rel_edges, W, b):
    del rel_edges
    B, N, D = node_fts.shape
    K = 128
    b2 = b.reshape(1, 1).astype(jnp.float32)
    p = jnp.pad(W.reshape(D, 1), ((0, 0), (0, D - 1)))

    TB = 4
    ranks, scores = pl.pallas_call(
        _score_rank_kernel,
        grid=(B // TB,),
        in_specs=[
            pl.BlockSpec((TB, N, D), lambda i: (i, 0, 0)),
            pl.BlockSpec((D, D), lambda i: (0, 0)),
            pl.BlockSpec((1, 1), lambda i: (0, 0)),
        ],
        out_specs=[
            pl.BlockSpec((TB, 1, N), lambda i: (i, 0, 0)),
            pl.BlockSpec((TB, 1, N), lambda i: (i, 0, 0)),
        ],
        out_shape=[
            jax.ShapeDtypeStruct((B, 1, N), jnp.int32),
            jax.ShapeDtypeStruct((B, 1, N), jnp.float32),
        ],
    )(node_fts, p, b2)
    ranks = ranks.reshape(B, N)
    scores = scores.reshape(B, N)

    sc_fn = functools.partial(
        pl.kernel,
        mesh=plsc.VectorSubcoreMesh(core_axis_name="c", subcore_axis_name="s"),
        out_type=[
            jax.ShapeDtypeStruct((B, K, D), jnp.float32),
            jax.ShapeDtypeStruct((B, K), jnp.float32),
        ],
        scratch_types=[
            pltpu.VMEM((N,), jnp.int32),
            pltpu.VMEM((N,), jnp.float32),
            pltpu.VMEM((K,), jnp.int32),
            pltpu.VMEM((K,), jnp.float32),
            pltpu.VMEM((K, D), jnp.float32),
            pltpu.SemaphoreType.DMA,
        ],
        compiler_params=pltpu.CompilerParams(needs_layout_passes=False),
    )(_sc_select_gather)
    gathered, value = sc_fn(ranks, scores, node_fts)

    SB = 16
    out = pl.pallas_call(
        _scale_kernel,
        grid=(B // SB,),
        in_specs=[
            pl.BlockSpec((SB, K, D), lambda i: (i, 0, 0)),
            pl.BlockSpec((K, D), lambda i: (0, 0)),
        ],
        out_specs=pl.BlockSpec((SB, K, D), lambda i: (i, 0, 0)),
        out_shape=jax.ShapeDtypeStruct((B, K, D), jnp.float32),
        compiler_params=pltpu.CompilerParams(
            dimension_semantics=("parallel",)),
    )(gathered, value)
    return out


# speedup vs baseline: 2.2744x; 1.0557x over previous
import functools

import jax
import jax.numpy as jnp
from jax import lax
from jax.experimental import pallas as pl
from jax.experimental.pallas import tpu as pltpu
from jax.experimental.pallas import tpu_sc as plsc


def _score_rank_kernel(x_ref, p_ref, b_ref, r_ref, s_ref):
    p_full = p_ref[...]
    bias = b_ref[0, 0]
    i_iota = jax.lax.broadcasted_iota(jnp.int32, (64, 512), 1)

    for t in range(x_ref.shape[0]):
        x = x_ref[t]
        y = jax.lax.dot_general(
            x.astype(jnp.bfloat16), p_full.astype(jnp.bfloat16),
            (((1,), (0,)), ((), ())),
            preferred_element_type=jnp.float32)
        wcol = y[:, 0:1]
        s_row = jax.nn.sigmoid(wcol.T + bias)
        s_col = s_row.T

        rank_row = jnp.zeros((1, 512), jnp.int32)
        for c in range(8):
            s_c = jax.lax.slice(s_col, (64 * c, 0), (64 * c + 64, 1))
            j_iota = (jax.lax.broadcasted_iota(jnp.int32, (64, 512), 0)
                      + (64 * c))
            cmp = (s_c > s_row) | ((s_c == s_row) & (j_iota < i_iota))
            rank_row = rank_row + jnp.sum(cmp.astype(jnp.int32), axis=0,
                                          keepdims=True)

        r_ref[t] = rank_row
        s_ref[t] = s_row


def _sc_select_gather(rank_hbm, score_hbm, node_hbm,
                      gath_hbm, val_hbm,
                      rank_v, score_v, idx_v, vals_v, rows_v, sem):
    nc = 2
    wid = lax.axis_index("s") * nc + lax.axis_index("c")
    for t in range(4):
        b = wid * 4 + t
        pltpu.sync_copy(rank_hbm.at[b], rank_v)
        pltpu.sync_copy(score_hbm.at[b], score_v)
        for c in range(32):
            r16 = rank_v[pl.ds(c * 16, 16)]
            s16 = score_v[pl.ds(c * 16, 16)]
            i16 = lax.iota(jnp.int32, 16) + (c * 16)
            msk = r16 < 128
            plsc.store_scatter(idx_v, [r16], i16, mask=msk)
            plsc.store_scatter(vals_v, [r16], s16, mask=msk)
        pltpu.async_copy(node_hbm.at[b].at[idx_v], rows_v, sem).wait()
        pltpu.sync_copy(rows_v, gath_hbm.at[b])
        pltpu.sync_copy(vals_v, val_hbm.at[b])


def _scale_kernel(g_ref, v_ref, o_ref):
    o_ref[...] = g_ref[...] * v_ref[...][None, :, :]


@jax.jit
def kernel(node_fts, rel_edges, W, b):
    del rel_edges
    B, N, D = node_fts.shape
    K = 128
    b2 = b.reshape(1, 1).astype(jnp.float32)
    p = jnp.pad(W.reshape(D, 1), ((0, 0), (0, D - 1)))

    TB = 8
    ranks, scores = pl.pallas_call(
        _score_rank_kernel,
        grid=(B // TB,),
        in_specs=[
            pl.BlockSpec((TB, N, D), lambda i: (i, 0, 0)),
            pl.BlockSpec((D, D), lambda i: (0, 0)),
            pl.BlockSpec((1, 1), lambda i: (0, 0)),
        ],
        out_specs=[
            pl.BlockSpec((TB, 1, N), lambda i: (i, 0, 0)),
            pl.BlockSpec((TB, 1, N), lambda i: (i, 0, 0)),
        ],
        out_shape=[
            jax.ShapeDtypeStruct((B, 1, N), jnp.int32),
            jax.ShapeDtypeStruct((B, 1, N), jnp.float32),
        ],
    )(node_fts, p, b2)
    ranks = ranks.reshape(B, N)
    scores = scores.reshape(B, N)

    sc_fn = functools.partial(
        pl.kernel,
        mesh=plsc.VectorSubcoreMesh(core_axis_name="c", subcore_axis_name="s"),
        out_type=[
            jax.ShapeDtypeStruct((B, K, D), jnp.float32),
            jax.ShapeDtypeStruct((B, K), jnp.float32),
        ],
        scratch_types=[
            pltpu.VMEM((N,), jnp.int32),
            pltpu.VMEM((N,), jnp.float32),
            pltpu.VMEM((K,), jnp.int32),
            pltpu.VMEM((K,), jnp.float32),
            pltpu.VMEM((K, D), jnp.float32),
            pltpu.SemaphoreType.DMA,
        ],
        compiler_params=pltpu.CompilerParams(needs_layout_passes=False),
    )(_sc_select_gather)
    gathered, value = sc_fn(ranks, scores, node_fts)

    SB = 16
    out = pl.pallas_call(
        _scale_kernel,
        grid=(B // SB,),
        in_specs=[
            pl.BlockSpec((SB, K, D), lambda i: (i, 0, 0)),
            pl.BlockSpec((K, D), lambda i: (0, 0)),
        ],
        out_specs=pl.BlockSpec((SB, K, D), lambda i: (i, 0, 0)),
        out_shape=jax.ShapeDtypeStruct((B, K, D), jnp.float32),
        compiler_params=pltpu.CompilerParams(
            dimension_semantics=("parallel",)),
    )(gathered, value)
    return out


# speedup vs baseline: 2.3641x; 1.0394x over previous
import functools

import jax
import jax.numpy as jnp
from jax import lax
from jax.experimental import pallas as pl
from jax.experimental.pallas import tpu as pltpu
from jax.experimental.pallas import tpu_sc as plsc


def _score_rank_kernel(x_ref, p_ref, b_ref, r_ref, s_ref):
    p_full = p_ref[...]
    bias = b_ref[0, 0]
    i_iota = jax.lax.broadcasted_iota(jnp.int32, (64, 512), 1)

    for t in range(x_ref.shape[0]):
        x = x_ref[t]
        y = jax.lax.dot_general(
            x.astype(jnp.bfloat16), p_full.astype(jnp.bfloat16),
            (((1,), (0,)), ((), ())),
            preferred_element_type=jnp.float32)
        wcol = y[:, 0:1]
        s_row = jax.nn.sigmoid(wcol.T + bias)
        s_col = s_row.T

        rank_row = jnp.zeros((1, 512), jnp.int32)
        for c in range(8):
            s_c = jax.lax.slice(s_col, (64 * c, 0), (64 * c + 64, 1))
            j_iota = (jax.lax.broadcasted_iota(jnp.int32, (64, 512), 0)
                      + (64 * c))
            cmp = (s_c > s_row) | ((s_c == s_row) & (j_iota < i_iota))
            rank_row = rank_row + jnp.sum(cmp.astype(jnp.int32), axis=0,
                                          keepdims=True)

        r_ref[pl.ds(t, 1), :] = rank_row
        s_ref[pl.ds(t, 1), :] = s_row


def _sc_select_gather(rank_hbm, score_hbm, node_hbm,
                      gath_hbm, val_hbm,
                      rank_v, score_v, idx_v, vals_v, rows_v, sem):
    nc = 2
    wid = lax.axis_index("s") * nc + lax.axis_index("c")
    for t in range(4):
        b = wid * 4 + t
        pltpu.sync_copy(rank_hbm.at[b], rank_v)
        pltpu.sync_copy(score_hbm.at[b], score_v)
        for c in range(32):
            r16 = rank_v[pl.ds(c * 16, 16)]
            s16 = score_v[pl.ds(c * 16, 16)]
            i16 = lax.iota(jnp.int32, 16) + (c * 16)
            msk = r16 < 128
            plsc.store_scatter(idx_v, [r16], i16, mask=msk)
            plsc.store_scatter(vals_v, [r16], s16, mask=msk)
        pltpu.async_copy(node_hbm.at[b].at[idx_v], rows_v, sem).wait()
        pltpu.sync_copy(rows_v, gath_hbm.at[b])
        pltpu.sync_copy(vals_v, val_hbm.at[b])


def _scale_kernel(g_ref, v_ref, o_ref):
    o_ref[...] = g_ref[...] * v_ref[...][None, :, :]


@jax.jit
def kernel(node_fts, rel_edges, W, b):
    del rel_edges
    B, N, D = node_fts.shape
    K = 128
    b2 = b.reshape(1, 1).astype(jnp.float32)
    p = jnp.pad(W.reshape(D, 1), ((0, 0), (0, D - 1)))

    TB = 8
    ranks, scores = pl.pallas_call(
        _score_rank_kernel,
        grid=(B // TB,),
        in_specs=[
            pl.BlockSpec((TB, N, D), lambda i: (i, 0, 0)),
            pl.BlockSpec((D, D), lambda i: (0, 0)),
            pl.BlockSpec((1, 1), lambda i: (0, 0)),
        ],
        out_specs=[
            pl.BlockSpec((TB, N), lambda i: (i, 0)),
            pl.BlockSpec((TB, N), lambda i: (i, 0)),
        ],
        out_shape=[
            jax.ShapeDtypeStruct((B, N), jnp.int32),
            jax.ShapeDtypeStruct((B, N), jnp.float32),
        ],
    )(node_fts, p, b2)

    sc_fn = functools.partial(
        pl.kernel,
        mesh=plsc.VectorSubcoreMesh(core_axis_name="c", subcore_axis_name="s"),
        out_type=[
            jax.ShapeDtypeStruct((B, K, D), jnp.float32),
            jax.ShapeDtypeStruct((B, K), jnp.float32),
        ],
        scratch_types=[
            pltpu.VMEM((N,), jnp.int32),
            pltpu.VMEM((N,), jnp.float32),
            pltpu.VMEM((K,), jnp.int32),
            pltpu.VMEM((K,), jnp.float32),
            pltpu.VMEM((K, D), jnp.float32),
            pltpu.SemaphoreType.DMA,
        ],
        compiler_params=pltpu.CompilerParams(needs_layout_passes=False),
    )(_sc_select_gather)
    gathered, value = sc_fn(ranks, scores, node_fts)

    SB = 16
    out = pl.pallas_call(
        _scale_kernel,
        grid=(B // SB,),
        in_specs=[
            pl.BlockSpec((SB, K, D), lambda i: (i, 0, 0)),
            pl.BlockSpec((K, D), lambda i: (0, 0)),
        ],
        out_specs=pl.BlockSpec((SB, K, D), lambda i: (i, 0, 0)),
        out_shape=jax.ShapeDtypeStruct((B, K, D), jnp.float32),
        compiler_params=pltpu.CompilerParams(
            dimension_semantics=("parallel",)),
    )(gathered, value)
    return out


# speedup vs baseline: 2.4195x; 1.0235x over previous
import functools

import jax
import jax.numpy as jnp
from jax import lax
from jax.experimental import pallas as pl
from jax.experimental.pallas import tpu as pltpu
from jax.experimental.pallas import tpu_sc as plsc


def _score_rank_kernel(x_ref, p_ref, b_ref, r_ref, s_ref):
    p_full = p_ref[...]
    bias = b_ref[0, 0]
    i_iota = jax.lax.broadcasted_iota(jnp.int32, (64, 512), 1)
    jlt = []
    for c in range(8):
        j_iota = jax.lax.broadcasted_iota(jnp.int32, (64, 512), 0) + (64 * c)
        jlt.append((j_iota < i_iota).astype(jnp.int32))

    for t in range(x_ref.shape[0]):
        x = x_ref[t]
        y = jax.lax.dot_general(
            x.astype(jnp.bfloat16), p_full.astype(jnp.bfloat16),
            (((1,), (0,)), ((), ())),
            preferred_element_type=jnp.float32)
        wcol = y[:, 0:1]
        s_row = jax.nn.sigmoid(wcol.T + bias)

        bits = jax.lax.bitcast_convert_type(s_row, jnp.int32)
        u_row = bits + bits
        u_col = u_row.T
        rank_row = jnp.zeros((1, 512), jnp.int32)
        for c in range(8):
            u_c = jax.lax.slice(u_col, (64 * c, 0), (64 * c + 64, 1))
            cmp = (u_c + jlt[c]) > u_row
            rank_row = rank_row + jnp.sum(cmp.astype(jnp.int32), axis=0,
                                          keepdims=True)

        r_ref[pl.ds(t, 1), :] = rank_row
        s_ref[pl.ds(t, 1), :] = s_row


def _sc_select_gather(rank_hbm, score_hbm, node_hbm,
                      gath_hbm, val_hbm,
                      rank_v, score_v, idx_v, vals_v, rows_v, sem):
    nc = 2
    wid = lax.axis_index("s") * nc + lax.axis_index("c")
    for t in range(4):
        b = wid * 4 + t
        pltpu.sync_copy(rank_hbm.at[b], rank_v)
        pltpu.sync_copy(score_hbm.at[b], score_v)
        for c in range(32):
            r16 = rank_v[pl.ds(c * 16, 16)]
            s16 = score_v[pl.ds(c * 16, 16)]
            i16 = lax.iota(jnp.int32, 16) + (c * 16)
            msk = r16 < 128
            plsc.store_scatter(idx_v, [r16], i16, mask=msk)
            plsc.store_scatter(vals_v, [r16], s16, mask=msk)
        pltpu.async_copy(node_hbm.at[b].at[idx_v], rows_v, sem).wait()
        pltpu.sync_copy(rows_v, gath_hbm.at[b])
        pltpu.sync_copy(vals_v, val_hbm.at[b])


def _scale_kernel(g_ref, v_ref, o_ref):
    o_ref[...] = g_ref[...] * v_ref[...][None, :, :]


@jax.jit
def kernel(node_fts, rel_edges, W, b):
    del rel_edges
    B, N, D = node_fts.shape
    K = 128
    b2 = b.reshape(1, 1).astype(jnp.float32)
    p = jnp.pad(W.reshape(D, 1), ((0, 0), (0, D - 1)))

    TB = 8
    ranks, scores = pl.pallas_call(
        _score_rank_kernel,
        grid=(B // TB,),
        in_specs=[
            pl.BlockSpec((TB, N, D), lambda i: (i, 0, 0)),
            pl.BlockSpec((D, D), lambda i: (0, 0)),
            pl.BlockSpec((1, 1), lambda i: (0, 0)),
        ],
        out_specs=[
            pl.BlockSpec((TB, N), lambda i: (i, 0)),
            pl.BlockSpec((TB, N), lambda i: (i, 0)),
        ],
        out_shape=[
            jax.ShapeDtypeStruct((B, N), jnp.int32),
            jax.ShapeDtypeStruct((B, N), jnp.float32),
        ],
    )(node_fts, p, b2)

    sc_fn = functools.partial(
        pl.kernel,
        mesh=plsc.VectorSubcoreMesh(core_axis_name="c", subcore_axis_name="s"),
        out_type=[
            jax.ShapeDtypeStruct((B, K, D), jnp.float32),
            jax.ShapeDtypeStruct((B, K), jnp.float32),
        ],
        scratch_types=[
            pltpu.VMEM((N,), jnp.int32),
            pltpu.VMEM((N,), jnp.float32),
            pltpu.VMEM((K,), jnp.int32),
            pltpu.VMEM((K,), jnp.float32),
            pltpu.VMEM((K, D), jnp.float32),
            pltpu.SemaphoreType.DMA,
        ],
        compiler_params=pltpu.CompilerParams(needs_layout_passes=False),
    )(_sc_select_gather)
    gathered, value = sc_fn(ranks, scores, node_fts)

    SB = 16
    out = pl.pallas_call(
        _scale_kernel,
        grid=(B // SB,),
        in_specs=[
            pl.BlockSpec((SB, K, D), lambda i: (i, 0, 0)),
            pl.BlockSpec((K, D), lambda i: (0, 0)),
        ],
        out_specs=pl.BlockSpec((SB, K, D), lambda i: (i, 0, 0)),
        out_shape=jax.ShapeDtypeStruct((B, K, D), jnp.float32),
        compiler_params=pltpu.CompilerParams(
            dimension_semantics=("parallel",)),
    )(gathered, value)
    return out
